# baseline (speedup 1.0000x reference)
import functools

import jax
import jax.numpy as jnp
from jax import lax
from jax.experimental import pallas as pl
from jax.experimental.pallas import tpu as pltpu
from jax.experimental.pallas import tpu_sc as plsc

N, M, K, C, H, FF, OUT = 65536, 4096, 32, 256, 8, 512, 256
D = C // H
SCALE = 1.0 / (D ** 0.5)
CP = 128
CW = 16

NC, NS = 2, 16
NW = NC * NS
B = M * K
RW = B // NW
CH = 128
NCH = RW // CH

@functools.cache
def _sc_gather_built(rows):
    RW = rows // NW
    NCH = RW // CH
    mesh = plsc.VectorSubcoreMesh(core_axis_name="c", subcore_axis_name="s",
                                  num_cores=NC, num_subcores=NS)

    @functools.partial(
        pl.kernel,
        out_type=[
            jax.ShapeDtypeStruct((rows, C), jnp.int32),
            jax.ShapeDtypeStruct((rows // 8, 128), jnp.int32),
        ],
        mesh=mesh,
        scratch_types=[
            pltpu.VMEM((CH,), jnp.int32), pltpu.VMEM((CH,), jnp.int32),
            pltpu.VMEM((CH, C), jnp.int32), pltpu.VMEM((CH, C), jnp.int32),
            pltpu.VMEM((CH, CP), jnp.int32), pltpu.VMEM((CH, CP), jnp.int32),
            pltpu.VMEM((CH // 8, 128), jnp.int32),
            pltpu.VMEM((CH // 8, 128), jnp.int32),
            pltpu.SemaphoreType.DMA, pltpu.SemaphoreType.DMA,
            pltpu.SemaphoreType.DMA, pltpu.SemaphoreType.DMA,
        ],
    )
    def body_fn(feat_hbm, coord_hbm, idx_hbm, feat_out, coord_out,
                idx0, idx1, f0, f1, cb0, cb1, cc0, cc1,
                g0, g1, w0, w1):
        wid = lax.axis_index("s") * NC + lax.axis_index("c")
        base = wid * RW
        idxs, fbs, cbs, ccs = (idx0, idx1), (f0, f1), (cb0, cb1), (cc0, cc1)
        gsems, wsems = (g0, g1), (w0, w1)

        def frows(c):
            return feat_out.at[pl.ds(base + c * CH, CH)]

        def crows(c):
            crow0 = pl.multiple_of((base + c * CH) // 8, CH // 8)
            return coord_out.at[pl.ds(crow0, CH // 8)]

        def start(c, b):
            @pl.when(c < NCH)
            def _():
                @pl.when(c >= 2)
                def _():
                    pltpu.make_async_copy(fbs[b], frows(c - 2), wsems[b]).wait()
                    pltpu.make_async_copy(ccs[b], crows(c - 2), wsems[b]).wait()
                pltpu.sync_copy(idx_hbm.at[pl.ds(base + c * CH, CH)], idxs[b])
                pltpu.async_copy(feat_hbm.at[idxs[b]], fbs[b], gsems[b])
                pltpu.async_copy(coord_hbm.at[idxs[b]], cbs[b], gsems[b])

        def finish(c, b):
            pltpu.make_async_copy(feat_hbm.at[idxs[b]], fbs[b], gsems[b]).wait()
            pltpu.make_async_copy(coord_hbm.at[idxs[b]], cbs[b], gsems[b]).wait()
            for r in range(CH):
                ccs[b][r // 8, pl.ds(CW * (r % 8), CW)] = cbs[b][r, pl.ds(0, CW)]
            pltpu.async_copy(fbs[b], frows(c), wsems[b])
            pltpu.async_copy(ccs[b], crows(c), wsems[b])

        start(0, 0)

        def body(j, carry):
            c = j * 2
            start(c + 1, 1)
            finish(c, 0)
            start(c + 2, 0)
            finish(c + 1, 1)
            return carry

        lax.fori_loop(0, NCH // 2, body, 0)
        pltpu.make_async_copy(fbs[0], frows(NCH - 2), wsems[0]).wait()
        pltpu.make_async_copy(ccs[0], crows(NCH - 2), wsems[0]).wait()
        pltpu.make_async_copy(fbs[1], frows(NCH - 1), wsems[1]).wait()
        pltpu.make_async_copy(ccs[1], crows(NCH - 1), wsems[1]).wait()

    return body_fn


def _sc_gather(feat, coord16, idx, rows=B):
    return _sc_gather_built(rows)(feat, coord16, idx)


BM = 64
GA = M // BM
BMK = BM * K


def _attn_block(kf_ref, kc_ref, qc_ref,
                kpwT_ref, kposb_ref, qpwT_ref, qposb_ref,
                wqT_ref, bq_ref, wkT_ref, bk_ref, wvT_ref, bv_ref,
                S_ref, ST_ref, woT_ref, bo_ref,
                l1T_ref, b1_ref, l2T_ref, b2_ref,
                new_ref, st_ref):
    i = pl.program_id(0)
    kfeat3 = kf_ref[...]
    rel = kc_ref[...] - qc_ref[...][:, None, :]
    kpos = jnp.maximum(
        jnp.dot(rel.reshape(BMK, CW), kpwT_ref[...],
                preferred_element_type=jnp.float32) + kposb_ref[...], 0.0)
    kffb = (kfeat3.reshape(BMK, C) + kpos).astype(jnp.bfloat16)
    qfeat = kfeat3.max(axis=1)
    qpos = jnp.maximum(
        jnp.dot(qc_ref[...], qpwT_ref[...],
                preferred_element_type=jnp.float32) + qposb_ref[...], 0.0)
    qf = qfeat + qpos
    q = jnp.dot(qf.astype(jnp.bfloat16), wqT_ref[...],
                preferred_element_type=jnp.float32) + bq_ref[...]
    q = q * SCALE
    k = jnp.dot(kffb, wkT_ref[...],
                preferred_element_type=jnp.float32) + bk_ref[...]
    v = jnp.dot(kffb, wvT_ref[...],
                preferred_element_type=jnp.float32) + bv_ref[...]
    prod = k.reshape(BM, K, C) * q[:, None, :]
    scores = jnp.dot(prod.reshape(BMK, C).astype(jnp.bfloat16), S_ref[...],
                     preferred_element_type=jnp.float32)
    s3 = scores.reshape(BM, K, H)
    mx = s3.max(axis=1, keepdims=True)
    e = jnp.exp(s3 - mx)
    a3 = e / e.sum(axis=1, keepdims=True)
    aexp = jnp.dot(a3.reshape(BMK, H).astype(jnp.bfloat16), ST_ref[...],
                   preferred_element_type=jnp.float32)
    o = (aexp * v).reshape(BM, K, C).sum(axis=1)
    attend = jnp.dot(o.astype(jnp.bfloat16), woT_ref[...],
                     preferred_element_type=jnp.float32) + bo_ref[...]
    hdn = jnp.maximum(
        jnp.dot(attend.astype(jnp.bfloat16), l1T_ref[...],
                preferred_element_type=jnp.float32) + b1_ref[...], 0.0)
    act = jnp.dot(hdn.astype(jnp.bfloat16), l2T_ref[...],
                  preferred_element_type=jnp.float32) + b2_ref[...]
    new = attend + act
    new_ref[...] = new

    @pl.when(i == 0)
    def _():
        st_ref[...] = jnp.zeros_like(st_ref)

    st_ref[0:1, :] += jnp.sum(new, axis=0, keepdims=True)
    st_ref[1:2, :] += jnp.sum(new * new, axis=0, keepdims=True)


def _bn1_block(new_ref, st_ref, g_ref, b_ref, owT_ref, ob_ref, y_ref, yst_ref):
    i = pl.program_id(0)
    st = st_ref[...]
    mu = st[0:1, :] * (1.0 / M)
    var = st[1:2, :] * (1.0 / M) - mu * mu
    s = g_ref[...] * lax.rsqrt(var + 1e-5)
    t = b_ref[...] - mu * s
    x = new_ref[...] * s + t
    y = jnp.dot(x.astype(jnp.bfloat16), owT_ref[...],
                preferred_element_type=jnp.float32) + ob_ref[...]
    y_ref[...] = y

    @pl.when(i == 0)
    def _():
        yst_ref[...] = jnp.zeros_like(yst_ref)

    yst_ref[0:1, :] += jnp.sum(y, axis=0, keepdims=True)
    yst_ref[1:2, :] += jnp.sum(y * y, axis=0, keepdims=True)


def _bn2_block(y_ref, yst_ref, g_ref, b_ref, out_ref):
    st = yst_ref[...]
    mu = st[0:1, :] * (1.0 / M)
    var = st[1:2, :] * (1.0 / M) - mu * mu
    s = g_ref[...] * lax.rsqrt(var + 1e-5)
    t = b_ref[...] - mu * s
    out_ref[...] = jnp.maximum(y_ref[...] * s + t, 0.0)


def _const2(shape):
    return pl.BlockSpec(shape, lambda i: (0, 0))


def kernel(voxel_features, voxel_coords, query_coords, key_indices,
           in_proj_w, in_proj_b, out_proj_w, out_proj_b,
           k_pos_w, k_pos_b, q_pos_w, q_pos_b,
           lin1_w, lin1_b, lin2_w, lin2_b,
           norm_g, norm_b, outl_w, outl_b, bn_out_g, bn_out_b):
    f32, bf16 = jnp.float32, jnp.bfloat16
    idx = key_indices.astype(jnp.int32).reshape(B)
    coord_pad = jnp.pad(voxel_coords, ((0, 0), (0, CP - 3)))
    qc_pad = jnp.pad(query_coords, ((0, 0), (0, CW - 3)))

    vf_i32 = lax.bitcast_convert_type(voxel_features, jnp.int32)
    cp_i32 = lax.bitcast_convert_type(coord_pad, jnp.int32)
    P = 1
    MH, BH = M // P, B // P
    gathered = [_sc_gather(vf_i32, cp_i32, idx[p * BH:(p + 1) * BH], BH)
                for p in range(P)]

    wq, wk, wv = in_proj_w[:C], in_proj_w[C:2 * C], in_proj_w[2 * C:]
    bq, bk, bv = (in_proj_b[:C], in_proj_b[C:2 * C], in_proj_b[2 * C:])
    kpwT = jnp.pad(k_pos_w, ((0, 0), (0, CW - 3))).T
    qpwT = jnp.pad(q_pos_w, ((0, 0), (0, CW - 3))).T
    S = jnp.repeat(jnp.eye(H, dtype=bf16), D, axis=0)
    ST = S.T
    row = lambda x: x.reshape(1, -1).astype(f32)

    def attn_half(kf3, kc3, qch):
        return pl.pallas_call(
            _attn_block,
            grid=(MH // BM,),
            in_specs=[
                pl.BlockSpec((BM, K, C), lambda i: (i, 0, 0)),
                pl.BlockSpec((BM, K, CW), lambda i: (i, 0, 0)),
                pl.BlockSpec((BM, CW), lambda i: (i, 0)),
                _const2((CW, C)), _const2((1, C)),
                _const2((CW, C)), _const2((1, C)),
                _const2((C, C)), _const2((1, C)),
                _const2((C, C)), _const2((1, C)),
                _const2((C, C)), _const2((1, C)),
                _const2((C, H)), _const2((H, C)),
                _const2((C, C)), _const2((1, C)),
                _const2((C, FF)), _const2((1, FF)),
                _const2((FF, C)), _const2((1, C)),
            ],
            out_specs=[
                pl.BlockSpec((BM, C), lambda i: (i, 0)),
                pl.BlockSpec((8, C), lambda i: (0, 0)),
            ],
            out_shape=[
                jax.ShapeDtypeStruct((MH, C), f32),
                jax.ShapeDtypeStruct((8, C), f32),
            ],
        )(kf3, kc3, qch,
          kpwT.astype(f32), row(k_pos_b), qpwT.astype(f32), row(q_pos_b),
          wq.T.astype(bf16), row(bq), wk.T.astype(bf16), row(bk),
          wv.T.astype(bf16), row(bv), S, ST,
          out_proj_w.T.astype(bf16), row(out_proj_b),
          lin1_w.T.astype(bf16), row(lin1_b), lin2_w.T.astype(bf16),
          row(lin2_b))

    news, sts = [], []
    for p in range(P):
        fi, ci = gathered[p]
        kf3 = lax.bitcast_convert_type(fi, f32).reshape(MH, K, C)
        kc3 = lax.bitcast_convert_type(ci, f32).reshape(MH, K, CW)
        new_h, st_h = attn_half(kf3, kc3, qc_pad[p * MH:(p + 1) * MH])
        news.append(new_h)
        sts.append(st_h)
    new = jnp.concatenate(news, axis=0)
    st = sts[0]
    for s_h in sts[1:]:
        st = st + s_h

    y, yst = pl.pallas_call(
        _bn1_block,
        grid=(GA,),
        in_specs=[
            pl.BlockSpec((BM, C), lambda i: (i, 0)),
            _const2((8, C)),
            _const2((1, C)), _const2((1, C)),
            _const2((C, OUT)), _const2((1, OUT)),
        ],
        out_specs=[
            pl.BlockSpec((BM, OUT), lambda i: (i, 0)),
            pl.BlockSpec((8, OUT), lambda i: (0, 0)),
        ],
        out_shape=[
            jax.ShapeDtypeStruct((M, OUT), f32),
            jax.ShapeDtypeStruct((8, OUT), f32),
        ],
    )(new, st, row(norm_g), row(norm_b), outl_w.T.astype(bf16), row(outl_b))

    out = pl.pallas_call(
        _bn2_block,
        grid=(GA,),
        in_specs=[
            pl.BlockSpec((BM, OUT), lambda i: (i, 0)),
            _const2((8, OUT)),
            _const2((1, OUT)), _const2((1, OUT)),
        ],
        out_specs=pl.BlockSpec((BM, OUT), lambda i: (i, 0)),
        out_shape=jax.ShapeDtypeStruct((M, OUT), f32),
    )(y, yst, row(bn_out_g), row(bn_out_b))
    return out


# speedup vs baseline: 1.2806x; 1.2806x over previous
import functools

import jax
import jax.numpy as jnp
from jax import lax
from jax.experimental import pallas as pl
from jax.experimental.pallas import tpu as pltpu
from jax.experimental.pallas import tpu_sc as plsc

N, M, K, C, H, FF, OUT = 65536, 4096, 32, 256, 8, 512, 256
D = C // H
SCALE = 1.0 / (D ** 0.5)
CP = 128
CW = 16

NC, NS = 2, 16
NW = NC * NS
B = M * K
RW = B // NW
CH = 128
NCH = RW // CH

@functools.cache
def _sc_gather_built(rows):
    RW = rows // NW
    NCH = RW // CH
    mesh = plsc.VectorSubcoreMesh(core_axis_name="c", subcore_axis_name="s",
                                  num_cores=NC, num_subcores=NS)

    @functools.partial(
        pl.kernel,
        out_type=[
            jax.ShapeDtypeStruct((rows, C), jnp.float32),
            jax.ShapeDtypeStruct((rows // 8, 128), jnp.float32),
        ],
        mesh=mesh,
        scratch_types=[
            pltpu.VMEM((CH,), jnp.int32), pltpu.VMEM((CH,), jnp.int32),
            pltpu.VMEM((CH, C), jnp.float32), pltpu.VMEM((CH, C), jnp.float32),
            pltpu.VMEM((CH, CP), jnp.float32), pltpu.VMEM((CH, CP), jnp.float32),
            pltpu.VMEM((CH // 8, 128), jnp.float32),
            pltpu.VMEM((CH // 8, 128), jnp.float32),
            pltpu.SemaphoreType.DMA, pltpu.SemaphoreType.DMA,
            pltpu.SemaphoreType.DMA, pltpu.SemaphoreType.DMA,
        ],
    )
    def body_fn(feat_hbm, coord_hbm, idx_hbm, feat_out, coord_out,
                idx0, idx1, f0, f1, cb0, cb1, cc0, cc1,
                g0, g1, w0, w1):
        wid = lax.axis_index("s") * NC + lax.axis_index("c")
        base = wid * RW
        idxs, fbs, cbs, ccs = (idx0, idx1), (f0, f1), (cb0, cb1), (cc0, cc1)
        gsems, wsems = (g0, g1), (w0, w1)

        def frows(c):
            return feat_out.at[pl.ds(base + c * CH, CH)]

        def crows(c):
            crow0 = pl.multiple_of((base + c * CH) // 8, CH // 8)
            return coord_out.at[pl.ds(crow0, CH // 8)]

        def start(c, b):
            @pl.when(c < NCH)
            def _():
                @pl.when(c >= 2)
                def _():
                    pltpu.make_async_copy(fbs[b], frows(c - 2), wsems[b]).wait()
                    pltpu.make_async_copy(ccs[b], crows(c - 2), wsems[b]).wait()
                pltpu.sync_copy(idx_hbm.at[pl.ds(base + c * CH, CH)], idxs[b])
                pltpu.async_copy(feat_hbm.at[idxs[b]], fbs[b], gsems[b])
                pltpu.async_copy(coord_hbm.at[idxs[b]], cbs[b], gsems[b])

        def finish(c, b):
            pltpu.make_async_copy(feat_hbm.at[idxs[b]], fbs[b], gsems[b]).wait()
            pltpu.make_async_copy(coord_hbm.at[idxs[b]], cbs[b], gsems[b]).wait()
            for r in range(CH):
                ccs[b][r // 8, pl.ds(CW * (r % 8), CW)] = cbs[b][r, pl.ds(0, CW)]
            pltpu.async_copy(fbs[b], frows(c), wsems[b])
            pltpu.async_copy(ccs[b], crows(c), wsems[b])

        start(0, 0)

        def body(j, carry):
            c = j * 2
            start(c + 1, 1)
            finish(c, 0)
            start(c + 2, 0)
            finish(c + 1, 1)
            return carry

        lax.fori_loop(0, NCH // 2, body, 0)
        pltpu.make_async_copy(fbs[0], frows(NCH - 2), wsems[0]).wait()
        pltpu.make_async_copy(ccs[0], crows(NCH - 2), wsems[0]).wait()
        pltpu.make_async_copy(fbs[1], frows(NCH - 1), wsems[1]).wait()
        pltpu.make_async_copy(ccs[1], crows(NCH - 1), wsems[1]).wait()

    return body_fn


def _sc_gather(feat, coord16, idx, rows=B):
    return _sc_gather_built(rows)(feat, coord16, idx)


BM = 64
GA = M // BM
BMK = BM * K


def _attn_block(kf_ref, kc_ref, qc_ref,
                kpwT_ref, kposb_ref, qpwT_ref, qposb_ref,
                wqT_ref, bq_ref, wkT_ref, bk_ref, wvT_ref, bv_ref,
                S_ref, ST_ref, woT_ref, bo_ref,
                l1T_ref, b1_ref, l2T_ref, b2_ref,
                new_ref, st_ref):
    i = pl.program_id(0)
    kfeat3 = kf_ref[...]
    rel = kc_ref[...] - qc_ref[...][:, None, :]
    kpos = jnp.maximum(
        jnp.dot(rel.reshape(BMK, CW), kpwT_ref[...],
                preferred_element_type=jnp.float32) + kposb_ref[...], 0.0)
    kffb = (kfeat3.reshape(BMK, C) + kpos).astype(jnp.bfloat16)
    qfeat = kfeat3.max(axis=1)
    qpos = jnp.maximum(
        jnp.dot(qc_ref[...], qpwT_ref[...],
                preferred_element_type=jnp.float32) + qposb_ref[...], 0.0)
    qf = qfeat + qpos
    q = jnp.dot(qf.astype(jnp.bfloat16), wqT_ref[...],
                preferred_element_type=jnp.float32) + bq_ref[...]
    q = q * SCALE
    k = jnp.dot(kffb, wkT_ref[...],
                preferred_element_type=jnp.float32) + bk_ref[...]
    v = jnp.dot(kffb, wvT_ref[...],
                preferred_element_type=jnp.float32) + bv_ref[...]
    prod = k.reshape(BM, K, C) * q[:, None, :]
    scores = jnp.dot(prod.reshape(BMK, C).astype(jnp.bfloat16), S_ref[...],
                     preferred_element_type=jnp.float32)
    s3 = scores.reshape(BM, K, H)
    mx = s3.max(axis=1, keepdims=True)
    e = jnp.exp(s3 - mx)
    a3 = e / e.sum(axis=1, keepdims=True)
    aexp = jnp.dot(a3.reshape(BMK, H).astype(jnp.bfloat16), ST_ref[...],
                   preferred_element_type=jnp.float32)
    o = (aexp * v).reshape(BM, K, C).sum(axis=1)
    attend = jnp.dot(o.astype(jnp.bfloat16), woT_ref[...],
                     preferred_element_type=jnp.float32) + bo_ref[...]
    hdn = jnp.maximum(
        jnp.dot(attend.astype(jnp.bfloat16), l1T_ref[...],
                preferred_element_type=jnp.float32) + b1_ref[...], 0.0)
    act = jnp.dot(hdn.astype(jnp.bfloat16), l2T_ref[...],
                  preferred_element_type=jnp.float32) + b2_ref[...]
    new = attend + act
    new_ref[...] = new

    @pl.when(i == 0)
    def _():
        st_ref[...] = jnp.zeros_like(st_ref)

    st_ref[0:1, :] += jnp.sum(new, axis=0, keepdims=True)
    st_ref[1:2, :] += jnp.sum(new * new, axis=0, keepdims=True)


def _bn1_block(new_ref, st_ref, g_ref, b_ref, owT_ref, ob_ref, y_ref, yst_ref):
    i = pl.program_id(0)
    st = st_ref[...]
    mu = st[0:1, :] * (1.0 / M)
    var = st[1:2, :] * (1.0 / M) - mu * mu
    s = g_ref[...] * lax.rsqrt(var + 1e-5)
    t = b_ref[...] - mu * s
    x = new_ref[...] * s + t
    y = jnp.dot(x.astype(jnp.bfloat16), owT_ref[...],
                preferred_element_type=jnp.float32) + ob_ref[...]
    y_ref[...] = y

    @pl.when(i == 0)
    def _():
        yst_ref[...] = jnp.zeros_like(yst_ref)

    yst_ref[0:1, :] += jnp.sum(y, axis=0, keepdims=True)
    yst_ref[1:2, :] += jnp.sum(y * y, axis=0, keepdims=True)


def _bn2_block(y_ref, yst_ref, g_ref, b_ref, out_ref):
    st = yst_ref[...]
    mu = st[0:1, :] * (1.0 / M)
    var = st[1:2, :] * (1.0 / M) - mu * mu
    s = g_ref[...] * lax.rsqrt(var + 1e-5)
    t = b_ref[...] - mu * s
    out_ref[...] = jnp.maximum(y_ref[...] * s + t, 0.0)


def _const2(shape):
    return pl.BlockSpec(shape, lambda i: (0, 0))


def kernel(voxel_features, voxel_coords, query_coords, key_indices,
           in_proj_w, in_proj_b, out_proj_w, out_proj_b,
           k_pos_w, k_pos_b, q_pos_w, q_pos_b,
           lin1_w, lin1_b, lin2_w, lin2_b,
           norm_g, norm_b, outl_w, outl_b, bn_out_g, bn_out_b):
    f32, bf16 = jnp.float32, jnp.bfloat16
    idx = key_indices.astype(jnp.int32).reshape(B)
    coord_pad = jnp.pad(voxel_coords, ((0, 0), (0, CP - 3)))
    qc_pad = jnp.pad(query_coords, ((0, 0), (0, CW - 3)))

    P = 1
    MH, BH = M // P, B // P
    gathered = [_sc_gather(voxel_features, coord_pad,
                           idx[p * BH:(p + 1) * BH], BH)
                for p in range(P)]

    wq, wk, wv = in_proj_w[:C], in_proj_w[C:2 * C], in_proj_w[2 * C:]
    bq, bk, bv = (in_proj_b[:C], in_proj_b[C:2 * C], in_proj_b[2 * C:])
    kpwT = jnp.pad(k_pos_w, ((0, 0), (0, CW - 3))).T
    qpwT = jnp.pad(q_pos_w, ((0, 0), (0, CW - 3))).T
    S = jnp.repeat(jnp.eye(H, dtype=bf16), D, axis=0)
    ST = S.T
    row = lambda x: x.reshape(1, -1).astype(f32)

    def attn_half(kf3, kc3, qch):
        return pl.pallas_call(
            _attn_block,
            grid=(MH // BM,),
            in_specs=[
                pl.BlockSpec((BM, K, C), lambda i: (i, 0, 0)),
                pl.BlockSpec((BM, K, CW), lambda i: (i, 0, 0)),
                pl.BlockSpec((BM, CW), lambda i: (i, 0)),
                _const2((CW, C)), _const2((1, C)),
                _const2((CW, C)), _const2((1, C)),
                _const2((C, C)), _const2((1, C)),
                _const2((C, C)), _const2((1, C)),
                _const2((C, C)), _const2((1, C)),
                _const2((C, H)), _const2((H, C)),
                _const2((C, C)), _const2((1, C)),
                _const2((C, FF)), _const2((1, FF)),
                _const2((FF, C)), _const2((1, C)),
            ],
            out_specs=[
                pl.BlockSpec((BM, C), lambda i: (i, 0)),
                pl.BlockSpec((8, C), lambda i: (0, 0)),
            ],
            out_shape=[
                jax.ShapeDtypeStruct((MH, C), f32),
                jax.ShapeDtypeStruct((8, C), f32),
            ],
        )(kf3, kc3, qch,
          kpwT.astype(f32), row(k_pos_b), qpwT.astype(f32), row(q_pos_b),
          wq.T.astype(bf16), row(bq), wk.T.astype(bf16), row(bk),
          wv.T.astype(bf16), row(bv), S, ST,
          out_proj_w.T.astype(bf16), row(out_proj_b),
          lin1_w.T.astype(bf16), row(lin1_b), lin2_w.T.astype(bf16),
          row(lin2_b))

    news, sts = [], []
    for p in range(P):
        fi, ci = gathered[p]
        kf3 = fi.reshape(MH, K, C)
        kc3 = ci.reshape(MH, K, CW)
        new_h, st_h = attn_half(kf3, kc3, qc_pad[p * MH:(p + 1) * MH])
        news.append(new_h)
        sts.append(st_h)
    new = jnp.concatenate(news, axis=0)
    st = sts[0]
    for s_h in sts[1:]:
        st = st + s_h

    y, yst = pl.pallas_call(
        _bn1_block,
        grid=(GA,),
        in_specs=[
            pl.BlockSpec((BM, C), lambda i: (i, 0)),
            _const2((8, C)),
            _const2((1, C)), _const2((1, C)),
            _const2((C, OUT)), _const2((1, OUT)),
        ],
        out_specs=[
            pl.BlockSpec((BM, OUT), lambda i: (i, 0)),
            pl.BlockSpec((8, OUT), lambda i: (0, 0)),
        ],
        out_shape=[
            jax.ShapeDtypeStruct((M, OUT), f32),
            jax.ShapeDtypeStruct((8, OUT), f32),
        ],
    )(new, st, row(norm_g), row(norm_b), outl_w.T.astype(bf16), row(outl_b))

    out = pl.pallas_call(
        _bn2_block,
        grid=(GA,),
        in_specs=[
            pl.BlockSpec((BM, OUT), lambda i: (i, 0)),
            _const2((8, OUT)),
            _const2((1, OUT)), _const2((1, OUT)),
        ],
        out_specs=pl.BlockSpec((BM, OUT), lambda i: (i, 0)),
        out_shape=jax.ShapeDtypeStruct((M, OUT), f32),
    )(y, yst, row(bn_out_g), row(bn_out_b))
    return out
